# baseline (speedup 1.0000x reference)
import functools

import jax
import jax.numpy as jnp
from jax import lax
from jax.experimental import pallas as pl
from jax.experimental.pallas import tpu as pltpu
from jax.experimental.pallas import tpu_sc as plsc

EMBED_DIM = 64
BATCH = 4096
SEQ_LEN = 500

NUM_WORKERS = 32
NPART = 32
PART_BATCH = BATCH // NPART
BATCH_PER_W = PART_BATCH // NUM_WORKERS
BPC = 2
CHUNK = BPC * SEQ_LEN
GATHER = 200
N_GATHER = CHUNK // GATHER
NBUF = 2
N_OUTER = BATCH_PER_W // (BPC * NBUF)


def _sc_gather(tokens_flat, table):
    mesh = plsc.VectorSubcoreMesh(core_axis_name="c", subcore_axis_name="s")

    @functools.partial(
        pl.kernel,
        mesh=mesh,
        out_type=jax.ShapeDtypeStruct(
            (PART_BATCH, SEQ_LEN, EMBED_DIM), jnp.float32
        ),
        scratch_types=[
            [pltpu.VMEM((CHUNK,), jnp.int32)] * NBUF,
            [pltpu.VMEM((CHUNK, EMBED_DIM), jnp.float32)] * NBUF,
            [pltpu.SemaphoreType.DMA] * NBUF,
            [pltpu.SemaphoreType.DMA] * NBUF,
        ],
        compiler_params=pltpu.CompilerParams(use_tc_tiling_on_sc=False),
    )
    def k(tok_hbm, table_hbm, out_hbm, idx_v, rows_v, sem_g, sem_o):
        wid = lax.axis_index("s") * 2 + lax.axis_index("c")
        w_batch = wid * BATCH_PER_W

        def body(t, carry):
            for b in range(NBUF):
                batch0 = w_batch + (t * NBUF + b) * BPC

                @pl.when(t > 0)
                def _drain_prev_writes(b=b):
                    for p in range(BPC):
                        pltpu.make_async_copy(
                            rows_v[b].at[pl.ds(p * SEQ_LEN, SEQ_LEN)],
                            out_hbm.at[0],
                            sem_o[b],
                        ).wait()

                pltpu.sync_copy(
                    tok_hbm.at[pl.ds(batch0 * SEQ_LEN, CHUNK)], idx_v[b]
                )
                for j in range(N_GATHER):
                    pltpu.async_copy(
                        table_hbm.at[idx_v[b].at[pl.ds(j * GATHER, GATHER)]],
                        rows_v[b].at[pl.ds(j * GATHER, GATHER)],
                        sem_g[b],
                    )
            for b in range(NBUF):
                batch0 = w_batch + (t * NBUF + b) * BPC
                for j in range(N_GATHER):
                    pltpu.make_async_copy(
                        table_hbm.at[idx_v[b].at[pl.ds(j * GATHER, GATHER)]],
                        rows_v[b].at[pl.ds(j * GATHER, GATHER)],
                        sem_g[b],
                    ).wait()
                for p in range(BPC):
                    pltpu.async_copy(
                        rows_v[b].at[pl.ds(p * SEQ_LEN, SEQ_LEN)],
                        out_hbm.at[batch0 + p],
                        sem_o[b],
                    )
            return carry

        lax.fori_loop(0, N_OUTER, body, 0)
        for b in range(NBUF):
            for p in range(BPC):
                pltpu.make_async_copy(
                    rows_v[b].at[pl.ds(p * SEQ_LEN, SEQ_LEN)],
                    out_hbm.at[0],
                    sem_o[b],
                ).wait()

    return k(tokens_flat, table)


def kernel(tokens, table):
    tokens_flat = tokens.reshape(BATCH * SEQ_LEN)
    n = PART_BATCH * SEQ_LEN
    parts = [
        _sc_gather(lax.dynamic_slice(tokens_flat, (i * n,), (n,)), table)
        for i in range(NPART)
    ]
    return jnp.concatenate(parts, axis=0)


# speedup vs baseline: 1.2027x; 1.2027x over previous
import functools

import jax
import jax.numpy as jnp
from jax import lax
from jax.experimental import pallas as pl
from jax.experimental.pallas import tpu as pltpu
from jax.experimental.pallas import tpu_sc as plsc

EMBED_DIM = 64
BATCH = 4096
SEQ_LEN = 500

NUM_WORKERS = 32
NPART = 16
PART_BATCH = BATCH // NPART
BATCH_PER_W = PART_BATCH // NUM_WORKERS
BPC = 2
CHUNK = BPC * SEQ_LEN
GATHER = 1000
N_GATHER = CHUNK // GATHER
NBUF = 2
N_OUTER = BATCH_PER_W // (BPC * NBUF)


def _sc_gather(tokens_flat, table):
    mesh = plsc.VectorSubcoreMesh(core_axis_name="c", subcore_axis_name="s")

    @functools.partial(
        pl.kernel,
        mesh=mesh,
        out_type=jax.ShapeDtypeStruct(
            (PART_BATCH, SEQ_LEN, EMBED_DIM), jnp.float32
        ),
        scratch_types=[
            [pltpu.VMEM((CHUNK,), jnp.int32)] * NBUF,
            [pltpu.VMEM((CHUNK, EMBED_DIM), jnp.float32)] * NBUF,
            [pltpu.SemaphoreType.DMA] * NBUF,
            [pltpu.SemaphoreType.DMA] * NBUF,
        ],
        compiler_params=pltpu.CompilerParams(use_tc_tiling_on_sc=False),
    )
    def k(tok_hbm, table_hbm, out_hbm, idx_v, rows_v, sem_g, sem_o):
        wid = lax.axis_index("s") * 2 + lax.axis_index("c")
        w_batch = wid * BATCH_PER_W

        def body(t, carry):
            for b in range(NBUF):
                batch0 = w_batch + (t * NBUF + b) * BPC

                @pl.when(t > 0)
                def _drain_prev_writes(b=b):
                    for p in range(BPC):
                        pltpu.make_async_copy(
                            rows_v[b].at[pl.ds(p * SEQ_LEN, SEQ_LEN)],
                            out_hbm.at[0],
                            sem_o[b],
                        ).wait()

                pltpu.sync_copy(
                    tok_hbm.at[pl.ds(batch0 * SEQ_LEN, CHUNK)], idx_v[b]
                )
                for j in range(N_GATHER):
                    pltpu.async_copy(
                        table_hbm.at[idx_v[b].at[pl.ds(j * GATHER, GATHER)]],
                        rows_v[b].at[pl.ds(j * GATHER, GATHER)],
                        sem_g[b],
                    )
            for b in range(NBUF):
                batch0 = w_batch + (t * NBUF + b) * BPC
                for j in range(N_GATHER):
                    pltpu.make_async_copy(
                        table_hbm.at[idx_v[b].at[pl.ds(j * GATHER, GATHER)]],
                        rows_v[b].at[pl.ds(j * GATHER, GATHER)],
                        sem_g[b],
                    ).wait()
                for p in range(BPC):
                    pltpu.async_copy(
                        rows_v[b].at[pl.ds(p * SEQ_LEN, SEQ_LEN)],
                        out_hbm.at[batch0 + p],
                        sem_o[b],
                    )
            return carry

        lax.fori_loop(0, N_OUTER, body, 0)
        for b in range(NBUF):
            for p in range(BPC):
                pltpu.make_async_copy(
                    rows_v[b].at[pl.ds(p * SEQ_LEN, SEQ_LEN)],
                    out_hbm.at[0],
                    sem_o[b],
                ).wait()

    return k(tokens_flat, table)


def kernel(tokens, table):
    tokens_flat = tokens.reshape(BATCH * SEQ_LEN)
    n = PART_BATCH * SEQ_LEN
    parts = [
        _sc_gather(lax.dynamic_slice(tokens_flat, (i * n,), (n,)), table)
        for i in range(NPART)
    ]
    return jnp.concatenate(parts, axis=0)
